# Initial kernel scaffold; baseline (speedup 1.0000x reference)
import jax
import jax.numpy as jnp
from jax.experimental import pallas as pl


def kernel(slab, s, a, r, d):
    raise NotImplementedError("write your pallas kernel here")


# speedup vs baseline: 1.3275x; 1.3275x over previous
import jax
import jax.numpy as jnp
from jax.experimental import pallas as pl
from jax.experimental.pallas import tpu as pltpu

_RNN_H = 64
_CT = 8


def _scan_kernel(s_ref, a_ref, r_ref, d_ref, wx_ref, wext_ref, wh_ref,
                 hn_ref, h_scr):
    tc = pl.program_id(1)

    @pl.when(tc == 0)
    def _():
        h_scr[...] = jnp.zeros_like(h_scr)

    wa = wext_ref[0:1, :]
    wr = wext_ref[1:2, :]
    wd = wext_ref[2:3, :]
    b = wext_ref[3:4, :]
    wh = wh_ref[...]
    h = h_scr[...]
    for i in range(_CT):
        px = (jnp.dot(s_ref[i], wx_ref[...], preferred_element_type=jnp.float32)
              + a_ref[i] * wa + r_ref[i] * wr + d_ref[i] * wd + b)
        h = jnp.tanh(px + jnp.dot(h, wh, preferred_element_type=jnp.float32))
        hn_ref[i] = h
    h_scr[...] = h


def _heads_kernel(h_ref, w1_ref, b1_ref, whd_ref, bh_ref, out_ref):
    y = jnp.maximum(
        jnp.dot(h_ref[...], w1_ref[...], preferred_element_type=jnp.float32)
        + b1_ref[0:1, :], 0.0)
    rv = jnp.dot(y, whd_ref[...], preferred_element_type=jnp.float32) + bh_ref[0:1, :]
    out_ref[...] = rv[:, :2]


def kernel(slab, s, a, r, d):
    s = jnp.asarray(s, jnp.float32)
    a = jnp.asarray(a, jnp.float32)[..., None]
    r = jnp.asarray(r, jnp.float32)[..., None]
    d = jnp.asarray(d, jnp.float32)[..., None]
    T, B, Dobs = s.shape
    BH = B // 2
    ct = _CT if T % _CT == 0 else 1

    wx = slab[:Dobs, :_RNN_H]
    wext = jnp.concatenate(
        [slab[Dobs:Dobs + 3, :_RNN_H], slab[512:513, :_RNN_H]], axis=0)
    wh = slab[128:128 + _RNN_H, :_RNN_H]
    w1 = slab[256:256 + _RNN_H, :]
    b1 = slab[513:514, :]
    whd = jnp.zeros((128, 8), jnp.float32)
    whd = whd.at[:, 0].set(slab[384:512, 64]).at[:, 1].set(slab[384:512, 65])
    bh = jnp.zeros((1, 8), jnp.float32)
    bh = bh.at[0, 0].set(slab[514, 64]).at[0, 1].set(slab[514, 65])

    hn = pl.pallas_call(
        _scan_kernel,
        out_shape=jax.ShapeDtypeStruct((T, B, _RNN_H), jnp.float32),
        grid=(2, T // ct),
        in_specs=[
            pl.BlockSpec((ct, BH, Dobs), lambda c, t: (t, c, 0)),
            pl.BlockSpec((ct, BH, 1), lambda c, t: (t, c, 0)),
            pl.BlockSpec((ct, BH, 1), lambda c, t: (t, c, 0)),
            pl.BlockSpec((ct, BH, 1), lambda c, t: (t, c, 0)),
            pl.BlockSpec((Dobs, _RNN_H), lambda c, t: (0, 0)),
            pl.BlockSpec((4, _RNN_H), lambda c, t: (0, 0)),
            pl.BlockSpec((_RNN_H, _RNN_H), lambda c, t: (0, 0)),
        ],
        out_specs=pl.BlockSpec((ct, BH, _RNN_H), lambda c, t: (t, c, 0)),
        scratch_shapes=[pltpu.VMEM((BH, _RNN_H), jnp.float32)],
        compiler_params=pltpu.CompilerParams(
            dimension_semantics=("parallel", "arbitrary")),
    )(s, a, r, d, wx, wext, wh)

    rows = T * B
    bm = 8192 if rows % 8192 == 0 else BH
    h2 = hn.reshape(rows, _RNN_H)
    rv = pl.pallas_call(
        _heads_kernel,
        out_shape=jax.ShapeDtypeStruct((rows, 2), jnp.float32),
        grid=(rows // bm,),
        in_specs=[
            pl.BlockSpec((bm, _RNN_H), lambda m: (m, 0)),
            pl.BlockSpec((_RNN_H, 128), lambda m: (0, 0)),
            pl.BlockSpec((1, 128), lambda m: (0, 0)),
            pl.BlockSpec((128, 8), lambda m: (0, 0)),
            pl.BlockSpec((1, 8), lambda m: (0, 0)),
        ],
        out_specs=pl.BlockSpec((bm, 2), lambda m: (m, 0)),
        compiler_params=pltpu.CompilerParams(
            dimension_semantics=("parallel",)),
    )(h2, w1, b1, whd, bh)

    rv = rv.reshape(T, B, 2)
    return rv[..., 0:1], rv[..., 1:2], hn


# speedup vs baseline: 1.8846x; 1.4196x over previous
import functools

import jax
import jax.numpy as jnp
from jax.experimental import pallas as pl
from jax.experimental.pallas import tpu as pltpu

_RNN_H = 64
_CT = 8
_HT = 32


def _scan_kernel(s_ref, ard_ref, wx_ref, ward_ref, wh_ref, hn_ref, h_scr,
                 *, ct):
    tc = pl.program_id(1)

    @pl.when(tc == 0)
    def _():
        h_scr[...] = jnp.zeros_like(h_scr)

    wh = wh_ref[...]
    h = h_scr[...]
    for i in range(ct):
        px = (jnp.dot(s_ref[i], wx_ref[...], preferred_element_type=jnp.float32)
              + jax.lax.dot_general(
                  ard_ref[i], ward_ref[...],
                  dimension_numbers=(((0,), (0,)), ((), ())),
                  preferred_element_type=jnp.float32))
        h = jnp.tanh(px + jnp.dot(h, wh, preferred_element_type=jnp.float32))
        hn_ref[i] = h
    h_scr[...] = h


def _heads_kernel(h_ref, w1_ref, b1_ref, whd_ref, bh_ref, ri_ref, lv_ref,
                  *, ht):
    for i in range(ht):
        y = jnp.maximum(
            jnp.dot(h_ref[i], w1_ref[...], preferred_element_type=jnp.float32)
            + b1_ref[0:1, :], 0.0)
        rv = (jnp.dot(y, whd_ref[...], preferred_element_type=jnp.float32)
              + bh_ref[0:1, :])
        ri_ref[i] = rv[:, 0:1]
        lv_ref[i] = rv[:, 1:2]


def kernel(slab, s, a, r, d):
    s = jnp.asarray(s, jnp.float32)
    T, B, Dobs = s.shape
    BH = B // 2
    ct = _CT if T % _CT == 0 else 1
    ht = _HT if T % _HT == 0 else 1

    ard = jnp.stack(
        [jnp.asarray(a, jnp.float32), jnp.asarray(r, jnp.float32),
         jnp.asarray(d, jnp.float32), jnp.ones((T, B), jnp.float32)], axis=1)

    wx = slab[:Dobs, :_RNN_H]
    ward = jnp.concatenate(
        [slab[Dobs:Dobs + 3, :_RNN_H], slab[512:513, :_RNN_H]], axis=0)
    wh = slab[128:128 + _RNN_H, :_RNN_H]
    w1 = slab[256:256 + _RNN_H, :]
    b1 = slab[513:514, :]
    whd = jnp.zeros((128, 8), jnp.float32)
    whd = whd.at[:, 0].set(slab[384:512, 64]).at[:, 1].set(slab[384:512, 65])
    bh = jnp.zeros((1, 8), jnp.float32)
    bh = bh.at[0, 0].set(slab[514, 64]).at[0, 1].set(slab[514, 65])

    hn = pl.pallas_call(
        functools.partial(_scan_kernel, ct=ct),
        out_shape=jax.ShapeDtypeStruct((T, B, _RNN_H), jnp.float32),
        grid=(2, T // ct),
        in_specs=[
            pl.BlockSpec((ct, BH, Dobs), lambda c, t: (t, c, 0)),
            pl.BlockSpec((ct, 4, BH), lambda c, t: (t, 0, c)),
            pl.BlockSpec((Dobs, _RNN_H), lambda c, t: (0, 0)),
            pl.BlockSpec((4, _RNN_H), lambda c, t: (0, 0)),
            pl.BlockSpec((_RNN_H, _RNN_H), lambda c, t: (0, 0)),
        ],
        out_specs=pl.BlockSpec((ct, BH, _RNN_H), lambda c, t: (t, c, 0)),
        scratch_shapes=[pltpu.VMEM((BH, _RNN_H), jnp.float32)],
        compiler_params=pltpu.CompilerParams(
            dimension_semantics=("parallel", "arbitrary")),
    )(s, ard, wx, ward, wh)

    ri, lv = pl.pallas_call(
        functools.partial(_heads_kernel, ht=ht),
        out_shape=(jax.ShapeDtypeStruct((T, B, 1), jnp.float32),
                   jax.ShapeDtypeStruct((T, B, 1), jnp.float32)),
        grid=(T // ht,),
        in_specs=[
            pl.BlockSpec((ht, B, _RNN_H), lambda t: (t, 0, 0)),
            pl.BlockSpec((_RNN_H, 128), lambda t: (0, 0)),
            pl.BlockSpec((1, 128), lambda t: (0, 0)),
            pl.BlockSpec((128, 8), lambda t: (0, 0)),
            pl.BlockSpec((1, 8), lambda t: (0, 0)),
        ],
        out_specs=(pl.BlockSpec((ht, B, 1), lambda t: (t, 0, 0)),
                   pl.BlockSpec((ht, B, 1), lambda t: (t, 0, 0))),
        compiler_params=pltpu.CompilerParams(
            dimension_semantics=("parallel",)),
    )(hn, w1, b1, whd, bh)

    return ri, lv, hn


# speedup vs baseline: 4.2818x; 2.2720x over previous
import functools

import jax
import jax.numpy as jnp
from jax.experimental import pallas as pl
from jax.experimental.pallas import tpu as pltpu

_RNN_H = 64
_CT = 8


def _fused_kernel(s_ref, ard_ref, wxt_ref, wardt_ref, wht_ref, w1bd_ref,
                  b1bd_ref, whdbd_ref, bhbd_ref, hn_ref, ri_ref, lv_ref,
                  h_scr, hall_scr, y_scr, *, ct):
    tc = pl.program_id(0)

    @pl.when(tc == 0)
    def _():
        h_scr[...] = jnp.zeros_like(h_scr)

    wht = wht_ref[...]
    h = h_scr[...]
    for i in range(ct):
        px = (jnp.dot(wxt_ref[...], s_ref[:, i, :],
                      preferred_element_type=jnp.float32)
              + jnp.dot(wardt_ref[...], ard_ref[i],
                        preferred_element_type=jnp.float32))
        h = jnp.tanh(px + jnp.dot(wht, h, preferred_element_type=jnp.float32))
        hn_ref[:, i, :] = h
        hall_scr[i * _RNN_H:(i + 1) * _RNN_H, :] = h
    h_scr[...] = h

    y_scr[...] = jnp.maximum(
        jnp.dot(w1bd_ref[...], hall_scr[...],
                preferred_element_type=jnp.float32) + b1bd_ref[...], 0.0)
    rv = (jnp.dot(whdbd_ref[...], y_scr[...],
                  preferred_element_type=jnp.float32) + bhbd_ref[...])
    for i in range(ct):
        ri_ref[i:i + 1, :] = rv[8 * i:8 * i + 1, :]
        lv_ref[i:i + 1, :] = rv[8 * i + 1:8 * i + 2, :]


def kernel(slab, s, a, r, d):
    s = jnp.asarray(s, jnp.float32)
    T, B, Dobs = s.shape
    ct = _CT if T % _CT == 0 else 1

    st = jnp.transpose(s, (2, 0, 1))

    ard = jnp.stack(
        [jnp.asarray(a, jnp.float32), jnp.asarray(r, jnp.float32),
         jnp.asarray(d, jnp.float32), jnp.ones((T, B), jnp.float32)], axis=1)

    wxt = slab[:Dobs, :_RNN_H].T
    wardt = jnp.concatenate(
        [slab[Dobs:Dobs + 3, :_RNN_H], slab[512:513, :_RNN_H]], axis=0).T
    wht = slab[128:128 + _RNN_H, :_RNN_H].T
    w1t = slab[256:256 + _RNN_H, :].T
    b1b = jnp.broadcast_to(slab[513:514, :].T, (128, B))
    whdt = jnp.zeros((8, 128), jnp.float32)
    whdt = whdt.at[0, :].set(slab[384:512, 64]).at[1, :].set(slab[384:512, 65])
    bhb = jnp.broadcast_to(
        jnp.zeros((8, 1), jnp.float32)
        .at[0, 0].set(slab[514, 64]).at[1, 0].set(slab[514, 65]), (8, B))

    eye_ct = jnp.eye(ct, dtype=jnp.float32)
    w1bd = jnp.kron(eye_ct, w1t)
    whdbd = jnp.kron(eye_ct, whdt)
    b1bd = jnp.tile(b1b, (ct, 1))
    bhbd = jnp.tile(bhb, (ct, 1))

    hn_t, ri, lv = pl.pallas_call(
        functools.partial(_fused_kernel, ct=ct),
        out_shape=(jax.ShapeDtypeStruct((_RNN_H, T, B), jnp.float32),
                   jax.ShapeDtypeStruct((T, B), jnp.float32),
                   jax.ShapeDtypeStruct((T, B), jnp.float32)),
        grid=(T // ct,),
        in_specs=[
            pl.BlockSpec((Dobs, ct, B), lambda t: (0, t, 0)),
            pl.BlockSpec((ct, 4, B), lambda t: (t, 0, 0)),
            pl.BlockSpec((_RNN_H, Dobs), lambda t: (0, 0)),
            pl.BlockSpec((_RNN_H, 4), lambda t: (0, 0)),
            pl.BlockSpec((_RNN_H, _RNN_H), lambda t: (0, 0)),
            pl.BlockSpec((ct * 128, ct * _RNN_H), lambda t: (0, 0)),
            pl.BlockSpec((ct * 128, B), lambda t: (0, 0)),
            pl.BlockSpec((ct * 8, ct * 128), lambda t: (0, 0)),
            pl.BlockSpec((ct * 8, B), lambda t: (0, 0)),
        ],
        out_specs=(pl.BlockSpec((_RNN_H, ct, B), lambda t: (0, t, 0)),
                   pl.BlockSpec((ct, B), lambda t: (t, 0)),
                   pl.BlockSpec((ct, B), lambda t: (t, 0))),
        scratch_shapes=[pltpu.VMEM((_RNN_H, B), jnp.float32),
                        pltpu.VMEM((ct * _RNN_H, B), jnp.float32),
                        pltpu.VMEM((ct * 128, B), jnp.float32)],
        compiler_params=pltpu.CompilerParams(
            dimension_semantics=("arbitrary",)),
    )(st, ard, wxt, wardt, wht, w1bd, b1bd, whdbd, bhbd)

    hn = jnp.transpose(hn_t, (1, 2, 0))
    return ri[..., None], lv[..., None], hn


# speedup vs baseline: 5.4823x; 1.2804x over previous
import functools

import jax
import jax.numpy as jnp
from jax.experimental import pallas as pl
from jax.experimental.pallas import tpu as pltpu

_RNN_H = 64
_CT = 8


def _fused_kernel(s_ref, ard_ref, wxt_ref, wardt_ref, wht_ref, w1bd_ref,
                  b1bd_ref, whdbd_ref, bhbd_ref, hn_ref, ri_ref, lv_ref,
                  h_scr, hall_scr, y_scr, *, ct):
    tc = pl.program_id(0)

    @pl.when(tc == 0)
    def _():
        h_scr[...] = jnp.zeros_like(h_scr)

    wht = wht_ref[...]
    h = h_scr[...]
    for i in range(ct):
        px = (jnp.dot(wxt_ref[...], s_ref[:, i, :],
                      preferred_element_type=jnp.float32)
              + jnp.dot(wardt_ref[...], ard_ref[i],
                        preferred_element_type=jnp.float32))
        h = jnp.tanh(px + jnp.dot(wht, h, preferred_element_type=jnp.float32))
        hn_ref[i] = h
        hall_scr[i * _RNN_H:(i + 1) * _RNN_H, :] = h
    h_scr[...] = h

    y_scr[...] = jnp.maximum(
        jnp.dot(w1bd_ref[...], hall_scr[...],
                preferred_element_type=jnp.float32) + b1bd_ref[...], 0.0)
    rv = (jnp.dot(whdbd_ref[...], y_scr[...],
                  preferred_element_type=jnp.float32) + bhbd_ref[...])
    for i in range(ct):
        ri_ref[i:i + 1, :] = rv[8 * i:8 * i + 1, :]
        lv_ref[i:i + 1, :] = rv[8 * i + 1:8 * i + 2, :]


def kernel(slab, s, a, r, d):
    s = jnp.asarray(s, jnp.float32)
    T, B, Dobs = s.shape
    ct = _CT if T % _CT == 0 else 1

    st = jnp.transpose(s, (2, 0, 1))

    ard = jnp.stack(
        [jnp.asarray(a, jnp.float32), jnp.asarray(r, jnp.float32),
         jnp.asarray(d, jnp.float32), jnp.ones((T, B), jnp.float32)], axis=1)

    wxt = slab[:Dobs, :_RNN_H].T
    wardt = jnp.concatenate(
        [slab[Dobs:Dobs + 3, :_RNN_H], slab[512:513, :_RNN_H]], axis=0).T
    wht = slab[128:128 + _RNN_H, :_RNN_H].T
    w1t = slab[256:256 + _RNN_H, :].T
    b1b = jnp.broadcast_to(slab[513:514, :].T, (128, B))
    whdt = jnp.zeros((8, 128), jnp.float32)
    whdt = whdt.at[0, :].set(slab[384:512, 64]).at[1, :].set(slab[384:512, 65])
    bhb = jnp.broadcast_to(
        jnp.zeros((8, 1), jnp.float32)
        .at[0, 0].set(slab[514, 64]).at[1, 0].set(slab[514, 65]), (8, B))

    eye_ct = jnp.eye(ct, dtype=jnp.float32)
    w1bd = jnp.kron(eye_ct, w1t)
    whdbd = jnp.kron(eye_ct, whdt)
    b1bd = jnp.tile(b1b, (ct, 1))
    bhbd = jnp.tile(bhb, (ct, 1))

    hn_t, ri, lv = pl.pallas_call(
        functools.partial(_fused_kernel, ct=ct),
        out_shape=(jax.ShapeDtypeStruct((T, _RNN_H, B), jnp.float32),
                   jax.ShapeDtypeStruct((T, B), jnp.float32),
                   jax.ShapeDtypeStruct((T, B), jnp.float32)),
        grid=(T // ct,),
        in_specs=[
            pl.BlockSpec((Dobs, ct, B), lambda t: (0, t, 0)),
            pl.BlockSpec((ct, 4, B), lambda t: (t, 0, 0)),
            pl.BlockSpec((_RNN_H, Dobs), lambda t: (0, 0)),
            pl.BlockSpec((_RNN_H, 4), lambda t: (0, 0)),
            pl.BlockSpec((_RNN_H, _RNN_H), lambda t: (0, 0)),
            pl.BlockSpec((ct * 128, ct * _RNN_H), lambda t: (0, 0)),
            pl.BlockSpec((ct * 128, B), lambda t: (0, 0)),
            pl.BlockSpec((ct * 8, ct * 128), lambda t: (0, 0)),
            pl.BlockSpec((ct * 8, B), lambda t: (0, 0)),
        ],
        out_specs=(pl.BlockSpec((ct, _RNN_H, B), lambda t: (t, 0, 0)),
                   pl.BlockSpec((ct, B), lambda t: (t, 0)),
                   pl.BlockSpec((ct, B), lambda t: (t, 0))),
        scratch_shapes=[pltpu.VMEM((_RNN_H, B), jnp.float32),
                        pltpu.VMEM((ct * _RNN_H, B), jnp.float32),
                        pltpu.VMEM((ct * 128, B), jnp.float32)],
        compiler_params=pltpu.CompilerParams(
            dimension_semantics=("arbitrary",)),
    )(st, ard, wxt, wardt, wht, w1bd, b1bd, whdbd, bhbd)

    hn = jnp.transpose(hn_t, (0, 2, 1))
    return ri[..., None], lv[..., None], hn


# speedup vs baseline: 6.6444x; 1.2120x over previous
import functools

import jax
import jax.numpy as jnp
from jax.experimental import pallas as pl
from jax.experimental.pallas import tpu as pltpu

_RNN_H = 64
_CT = 16


def _heads_rv(whdt_ref, bhb_ref, y_scr, ri_ref, lv_ref, i):
    rv = (jnp.dot(whdt_ref[...], y_scr[i * 128:(i + 1) * 128, :],
                  preferred_element_type=jnp.float32) + bhb_ref[...])
    ri_ref[i:i + 1, :] = rv[0:1, :]
    lv_ref[i:i + 1, :] = rv[1:2, :]


def _fused_kernel(s_ref, ard_ref, wxt_ref, wardt_ref, wht_ref, w1t_ref,
                  b1b_ref, whdt_ref, bhb_ref, hn_ref, ri_ref, lv_ref,
                  h_scr, y_scr, *, ct):
    tc = pl.program_id(0)

    @pl.when(tc == 0)
    def _():
        h_scr[...] = jnp.zeros_like(h_scr)

    wht = wht_ref[...]
    h = h_scr[...]
    for i in range(ct):
        if i >= 1:
            y_scr[(i - 1) * 128:i * 128, :] = jnp.maximum(
                jnp.dot(w1t_ref[...], h, preferred_element_type=jnp.float32)
                + b1b_ref[...], 0.0)
        if i >= 2:
            _heads_rv(whdt_ref, bhb_ref, y_scr, ri_ref, lv_ref, i - 2)
        px = (jnp.dot(wxt_ref[...], s_ref[:, i, :],
                      preferred_element_type=jnp.float32)
              + jnp.dot(wardt_ref[...], ard_ref[i],
                        preferred_element_type=jnp.float32))
        h = jnp.tanh(px + jnp.dot(wht, h, preferred_element_type=jnp.float32))
        hn_ref[i] = h
    h_scr[...] = h

    y_scr[(ct - 1) * 128:ct * 128, :] = jnp.maximum(
        jnp.dot(w1t_ref[...], h, preferred_element_type=jnp.float32)
        + b1b_ref[...], 0.0)
    for i in range(max(ct - 2, 0), ct):
        _heads_rv(whdt_ref, bhb_ref, y_scr, ri_ref, lv_ref, i)


def kernel(slab, s, a, r, d):
    s = jnp.asarray(s, jnp.float32)
    T, B, Dobs = s.shape
    ct = _CT if T % _CT == 0 else 1

    st = jnp.transpose(s, (2, 0, 1))

    ard = jnp.stack(
        [jnp.asarray(a, jnp.float32), jnp.asarray(r, jnp.float32),
         jnp.asarray(d, jnp.float32), jnp.ones((T, B), jnp.float32)], axis=1)

    wxt = slab[:Dobs, :_RNN_H].T
    wardt = jnp.concatenate(
        [slab[Dobs:Dobs + 3, :_RNN_H], slab[512:513, :_RNN_H]], axis=0).T
    wht = slab[128:128 + _RNN_H, :_RNN_H].T
    w1t = slab[256:256 + _RNN_H, :].T
    b1b = jnp.broadcast_to(slab[513:514, :].T, (128, B))
    whdt = jnp.zeros((_RNN_H, 128), jnp.float32)
    whdt = whdt.at[0, :].set(slab[384:512, 64]).at[1, :].set(slab[384:512, 65])
    bhb = jnp.broadcast_to(
        jnp.zeros((_RNN_H, 1), jnp.float32)
        .at[0, 0].set(slab[514, 64]).at[1, 0].set(slab[514, 65]), (_RNN_H, B))

    hn_t, ri, lv = pl.pallas_call(
        functools.partial(_fused_kernel, ct=ct),
        out_shape=(jax.ShapeDtypeStruct((T, _RNN_H, B), jnp.float32),
                   jax.ShapeDtypeStruct((T, B), jnp.float32),
                   jax.ShapeDtypeStruct((T, B), jnp.float32)),
        grid=(T // ct,),
        in_specs=[
            pl.BlockSpec((Dobs, ct, B), lambda t: (0, t, 0)),
            pl.BlockSpec((ct, 4, B), lambda t: (t, 0, 0)),
            pl.BlockSpec((_RNN_H, Dobs), lambda t: (0, 0)),
            pl.BlockSpec((_RNN_H, 4), lambda t: (0, 0)),
            pl.BlockSpec((_RNN_H, _RNN_H), lambda t: (0, 0)),
            pl.BlockSpec((128, _RNN_H), lambda t: (0, 0)),
            pl.BlockSpec((128, B), lambda t: (0, 0)),
            pl.BlockSpec((_RNN_H, 128), lambda t: (0, 0)),
            pl.BlockSpec((_RNN_H, B), lambda t: (0, 0)),
        ],
        out_specs=(pl.BlockSpec((ct, _RNN_H, B), lambda t: (t, 0, 0)),
                   pl.BlockSpec((ct, B), lambda t: (t, 0)),
                   pl.BlockSpec((ct, B), lambda t: (t, 0))),
        scratch_shapes=[pltpu.VMEM((_RNN_H, B), jnp.float32),
                        pltpu.VMEM((ct * 128, B), jnp.float32)],
        compiler_params=pltpu.CompilerParams(
            dimension_semantics=("arbitrary",)),
    )(st, ard, wxt, wardt, wht, w1t, b1b, whdt, bhb)

    hn = jnp.transpose(hn_t, (0, 2, 1))
    return ri[..., None], lv[..., None], hn


# speedup vs baseline: 7.0094x; 1.0549x over previous
import functools

import jax
import jax.numpy as jnp
from jax.experimental import pallas as pl
from jax.experimental.pallas import tpu as pltpu

_RNN_H = 64
_CT = 32


def _heads_rv(whdt_ref, bhb_ref, y_scr, ri_ref, lv_ref, i):
    rv = (jnp.dot(whdt_ref[...], y_scr[i * 128:(i + 1) * 128, :],
                  preferred_element_type=jnp.float32) + bhb_ref[...])
    ri_ref[i:i + 1, :] = rv[0:1, :]
    lv_ref[i:i + 1, :] = rv[1:2, :]


def _fused_kernel(s_ref, ard_ref, wxt_ref, wardt_ref, wht_ref, w1t_ref,
                  b1b_ref, whdt_ref, bhb_ref, hn_ref, ri_ref, lv_ref,
                  h_scr, y_scr, *, ct):
    tc = pl.program_id(0)

    @pl.when(tc == 0)
    def _():
        h_scr[...] = jnp.zeros_like(h_scr)

    wht = wht_ref[...]
    h = h_scr[...]
    for i in range(ct):
        if i >= 1:
            y_scr[(i - 1) * 128:i * 128, :] = jnp.maximum(
                jnp.dot(w1t_ref[...], h, preferred_element_type=jnp.float32)
                + b1b_ref[...], 0.0)
        if i >= 2:
            _heads_rv(whdt_ref, bhb_ref, y_scr, ri_ref, lv_ref, i - 2)
        px = (jnp.dot(wxt_ref[...], s_ref[:, i, :],
                      preferred_element_type=jnp.float32)
              + jnp.dot(wardt_ref[...], ard_ref[i],
                        preferred_element_type=jnp.float32))
        h = jnp.tanh(px + jnp.dot(wht, h, preferred_element_type=jnp.float32))
        hn_ref[i] = h
    h_scr[...] = h

    y_scr[(ct - 1) * 128:ct * 128, :] = jnp.maximum(
        jnp.dot(w1t_ref[...], h, preferred_element_type=jnp.float32)
        + b1b_ref[...], 0.0)
    for i in range(max(ct - 2, 0), ct):
        _heads_rv(whdt_ref, bhb_ref, y_scr, ri_ref, lv_ref, i)


def kernel(slab, s, a, r, d):
    s = jnp.asarray(s, jnp.float32)
    T, B, Dobs = s.shape
    ct = _CT if T % _CT == 0 else 1

    st = jnp.transpose(s, (2, 0, 1))

    ard = jnp.stack(
        [jnp.asarray(a, jnp.float32), jnp.asarray(r, jnp.float32),
         jnp.asarray(d, jnp.float32), jnp.ones((T, B), jnp.float32)], axis=1)

    wxt = slab[:Dobs, :_RNN_H].T
    wardt = jnp.concatenate(
        [slab[Dobs:Dobs + 3, :_RNN_H], slab[512:513, :_RNN_H]], axis=0).T
    wht = slab[128:128 + _RNN_H, :_RNN_H].T
    w1t = slab[256:256 + _RNN_H, :].T
    b1b = jnp.broadcast_to(slab[513:514, :].T, (128, B))
    whdt = jnp.zeros((_RNN_H, 128), jnp.float32)
    whdt = whdt.at[0, :].set(slab[384:512, 64]).at[1, :].set(slab[384:512, 65])
    bhb = jnp.broadcast_to(
        jnp.zeros((_RNN_H, 1), jnp.float32)
        .at[0, 0].set(slab[514, 64]).at[1, 0].set(slab[514, 65]), (_RNN_H, B))

    hn_t, ri, lv = pl.pallas_call(
        functools.partial(_fused_kernel, ct=ct),
        out_shape=(jax.ShapeDtypeStruct((T, _RNN_H, B), jnp.float32),
                   jax.ShapeDtypeStruct((T, B), jnp.float32),
                   jax.ShapeDtypeStruct((T, B), jnp.float32)),
        grid=(T // ct,),
        in_specs=[
            pl.BlockSpec((Dobs, ct, B), lambda t: (0, t, 0)),
            pl.BlockSpec((ct, 4, B), lambda t: (t, 0, 0)),
            pl.BlockSpec((_RNN_H, Dobs), lambda t: (0, 0)),
            pl.BlockSpec((_RNN_H, 4), lambda t: (0, 0)),
            pl.BlockSpec((_RNN_H, _RNN_H), lambda t: (0, 0)),
            pl.BlockSpec((128, _RNN_H), lambda t: (0, 0)),
            pl.BlockSpec((128, B), lambda t: (0, 0)),
            pl.BlockSpec((_RNN_H, 128), lambda t: (0, 0)),
            pl.BlockSpec((_RNN_H, B), lambda t: (0, 0)),
        ],
        out_specs=(pl.BlockSpec((ct, _RNN_H, B), lambda t: (t, 0, 0)),
                   pl.BlockSpec((ct, B), lambda t: (t, 0)),
                   pl.BlockSpec((ct, B), lambda t: (t, 0))),
        scratch_shapes=[pltpu.VMEM((_RNN_H, B), jnp.float32),
                        pltpu.VMEM((ct * 128, B), jnp.float32)],
        compiler_params=pltpu.CompilerParams(
            dimension_semantics=("arbitrary",)),
    )(st, ard, wxt, wardt, wht, w1t, b1b, whdt, bhb)

    hn = jnp.transpose(hn_t, (0, 2, 1))
    return ri[..., None], lv[..., None], hn


# speedup vs baseline: 7.0368x; 1.0039x over previous
import functools

import jax
import jax.numpy as jnp
from jax.experimental import pallas as pl
from jax.experimental.pallas import tpu as pltpu

_RNN_H = 64
_CT = 32


_TA = (((0,), (0,)), ((), ()))


def _dott(w_ref, x):
    return jax.lax.dot_general(w_ref[...], x, dimension_numbers=_TA,
                               preferred_element_type=jnp.float32)


def _heads_rv(whd_ref, bhb_ref, y_scr, ri_ref, lv_ref, i):
    rv = _dott(whd_ref, y_scr[i * 128:(i + 1) * 128, :]) + bhb_ref[...]
    ri_ref[i:i + 1, :] = rv[0:1, :]
    lv_ref[i:i + 1, :] = rv[1:2, :]


def _fused_kernel(s_ref, ard_ref, wx_ref, ward_ref, wh_ref, w1_ref,
                  b1b_ref, whd_ref, bhb_ref, hn_ref, ri_ref, lv_ref,
                  h_scr, y_scr, *, ct):
    tc = pl.program_id(0)

    @pl.when(tc == 0)
    def _():
        h_scr[...] = jnp.zeros_like(h_scr)

    h = h_scr[...]
    for i in range(ct):
        if i >= 1:
            y_scr[(i - 1) * 128:i * 128, :] = jnp.maximum(
                _dott(w1_ref, h) + b1b_ref[...], 0.0)
        if i >= 2:
            _heads_rv(whd_ref, bhb_ref, y_scr, ri_ref, lv_ref, i - 2)
        px = _dott(wx_ref, s_ref[:, i, :]) + _dott(ward_ref, ard_ref[i])
        h = jnp.tanh(px + _dott(wh_ref, h))
        hn_ref[i] = h
    h_scr[...] = h

    y_scr[(ct - 1) * 128:ct * 128, :] = jnp.maximum(
        _dott(w1_ref, h) + b1b_ref[...], 0.0)
    for i in range(max(ct - 2, 0), ct):
        _heads_rv(whd_ref, bhb_ref, y_scr, ri_ref, lv_ref, i)


def kernel(slab, s, a, r, d):
    s = jnp.asarray(s, jnp.float32)
    T, B, Dobs = s.shape
    ct = _CT if T % _CT == 0 else 1

    st = jnp.transpose(s, (2, 0, 1))

    ard = jnp.stack(
        [jnp.asarray(a, jnp.float32), jnp.asarray(r, jnp.float32),
         jnp.asarray(d, jnp.float32), jnp.ones((T, B), jnp.float32)], axis=1)

    wx = slab[:Dobs, :_RNN_H]
    ward = jnp.concatenate(
        [slab[Dobs:Dobs + 3, :_RNN_H], slab[512:513, :_RNN_H]], axis=0)
    wh = slab[128:128 + _RNN_H, :_RNN_H]
    w1 = slab[256:256 + _RNN_H, :]
    b1b = jnp.broadcast_to(slab[513:514, :].T, (128, B))
    whd = slab[384:512, 64:128]
    bhb = jnp.broadcast_to(slab[514:515, 64:128].T, (_RNN_H, B))

    hn_t, ri, lv = pl.pallas_call(
        functools.partial(_fused_kernel, ct=ct),
        out_shape=(jax.ShapeDtypeStruct((T, _RNN_H, B), jnp.float32),
                   jax.ShapeDtypeStruct((T, B), jnp.float32),
                   jax.ShapeDtypeStruct((T, B), jnp.float32)),
        grid=(T // ct,),
        in_specs=[
            pl.BlockSpec((Dobs, ct, B), lambda t: (0, t, 0)),
            pl.BlockSpec((ct, 4, B), lambda t: (t, 0, 0)),
            pl.BlockSpec((Dobs, _RNN_H), lambda t: (0, 0)),
            pl.BlockSpec((4, _RNN_H), lambda t: (0, 0)),
            pl.BlockSpec((_RNN_H, _RNN_H), lambda t: (0, 0)),
            pl.BlockSpec((_RNN_H, 128), lambda t: (0, 0)),
            pl.BlockSpec((128, B), lambda t: (0, 0)),
            pl.BlockSpec((128, _RNN_H), lambda t: (0, 0)),
            pl.BlockSpec((_RNN_H, B), lambda t: (0, 0)),
        ],
        out_specs=(pl.BlockSpec((ct, _RNN_H, B), lambda t: (t, 0, 0)),
                   pl.BlockSpec((ct, B), lambda t: (t, 0)),
                   pl.BlockSpec((ct, B), lambda t: (t, 0))),
        scratch_shapes=[pltpu.VMEM((_RNN_H, B), jnp.float32),
                        pltpu.VMEM((ct * 128, B), jnp.float32)],
        compiler_params=pltpu.CompilerParams(
            dimension_semantics=("arbitrary",)),
    )(st, ard, wx, ward, wh, w1, b1b, whd, bhb)

    hn = jnp.transpose(hn_t, (0, 2, 1))
    return ri[..., None], lv[..., None], hn


# speedup vs baseline: 8.1597x; 1.1596x over previous
import functools

import jax
import jax.numpy as jnp
from jax.experimental import pallas as pl
from jax.experimental.pallas import tpu as pltpu

_RNN_H = 64
_CT = 32

_TA = (((0,), (0,)), ((), ()))


def _dott(w, x):
    return jax.lax.dot_general(w, x, dimension_numbers=_TA,
                               preferred_element_type=jnp.float32)


def _fused_kernel(s_ref, a_ref, r_ref, d_ref, slab_ref, hn_ref, ri_ref,
                  lv_ref, h_scr, y_scr, *, ct):
    tc = pl.program_id(0)

    @pl.when(tc == 0)
    def _():
        h_scr[...] = jnp.zeros_like(h_scr)

    B = s_ref.shape[2]
    dobs = s_ref.shape[0]
    ones1 = jnp.ones((1, B), jnp.float32)

    wx = slab_ref[0:dobs, 0:_RNN_H]
    ward = jnp.concatenate(
        [slab_ref[dobs:dobs + 3, 0:_RNN_H],
         slab_ref[512:513, 0:_RNN_H]], axis=0)
    wh = slab_ref[128:128 + _RNN_H, 0:_RNN_H]
    w1a = jnp.concatenate(
        [slab_ref[256:256 + _RNN_H, :], slab_ref[513:514, :]], axis=0)
    whda = jnp.concatenate(
        [slab_ref[384:512, 64:128], slab_ref[514:515, 64:128]], axis=0)

    def heads_rv(i):
        yaug = jnp.concatenate(
            [y_scr[i * 128:(i + 1) * 128, :], ones1], axis=0)
        rv = _dott(whda, yaug)
        ri_ref[i:i + 1, :] = rv[0:1, :]
        lv_ref[i:i + 1, :] = rv[1:2, :]

    h = h_scr[...]
    haug = jnp.concatenate([h, ones1], axis=0)
    for i in range(ct):
        if i >= 1:
            y_scr[(i - 1) * 128:i * 128, :] = jnp.maximum(_dott(w1a, haug), 0.0)
        if i >= 2:
            heads_rv(i - 2)
        ard = jnp.concatenate(
            [a_ref[i:i + 1, :], r_ref[i:i + 1, :], d_ref[i:i + 1, :], ones1],
            axis=0)
        px = _dott(wx, s_ref[:, i, :]) + _dott(ward, ard)
        h = jnp.tanh(px + _dott(wh, h))
        hn_ref[i] = h
        haug = jnp.concatenate([h, ones1], axis=0)
    h_scr[...] = h

    y_scr[(ct - 1) * 128:ct * 128, :] = jnp.maximum(_dott(w1a, haug), 0.0)
    for i in range(max(ct - 2, 0), ct):
        heads_rv(i)


def kernel(slab, s, a, r, d):
    s = jnp.asarray(s, jnp.float32)
    a = jnp.asarray(a, jnp.float32)
    r = jnp.asarray(r, jnp.float32)
    d = jnp.asarray(d, jnp.float32)
    T, B, Dobs = s.shape
    ct = _CT if T % _CT == 0 else 1

    st = jnp.transpose(s, (2, 0, 1))

    hn_t, ri, lv = pl.pallas_call(
        functools.partial(_fused_kernel, ct=ct),
        out_shape=(jax.ShapeDtypeStruct((T, _RNN_H, B), jnp.float32),
                   jax.ShapeDtypeStruct((T, B), jnp.float32),
                   jax.ShapeDtypeStruct((T, B), jnp.float32)),
        grid=(T // ct,),
        in_specs=[
            pl.BlockSpec((Dobs, ct, B), lambda t: (0, t, 0)),
            pl.BlockSpec((ct, B), lambda t: (t, 0)),
            pl.BlockSpec((ct, B), lambda t: (t, 0)),
            pl.BlockSpec((ct, B), lambda t: (t, 0)),
            pl.BlockSpec((520, 128), lambda t: (0, 0)),
        ],
        out_specs=(pl.BlockSpec((ct, _RNN_H, B), lambda t: (t, 0, 0)),
                   pl.BlockSpec((ct, B), lambda t: (t, 0)),
                   pl.BlockSpec((ct, B), lambda t: (t, 0))),
        scratch_shapes=[pltpu.VMEM((_RNN_H, B), jnp.float32),
                        pltpu.VMEM((ct * 128, B), jnp.float32)],
        compiler_params=pltpu.CompilerParams(
            dimension_semantics=("arbitrary",)),
    )(st, a, r, d, slab)

    hn = jnp.transpose(hn_t, (0, 2, 1))
    return ri[..., None], lv[..., None], hn


# speedup vs baseline: 8.3111x; 1.0186x over previous
import functools

import jax
import jax.numpy as jnp
from jax.experimental import pallas as pl
from jax.experimental.pallas import tpu as pltpu

_RNN_H = 64
_CT = 64

_TA = (((0,), (0,)), ((), ()))


def _dott(w, x):
    return jax.lax.dot_general(w, x, dimension_numbers=_TA,
                               preferred_element_type=jnp.float32)


def _fused_kernel(s_ref, a_ref, r_ref, d_ref, slab_ref, hn_ref, ri_ref,
                  lv_ref, h_scr, y_scr, *, ct):
    tc = pl.program_id(0)

    @pl.when(tc == 0)
    def _():
        h_scr[...] = jnp.zeros_like(h_scr)

    B = s_ref.shape[2]
    dobs = s_ref.shape[0]
    ones1 = jnp.ones((1, B), jnp.float32)

    wx = slab_ref[0:dobs, 0:_RNN_H]
    ward = jnp.concatenate(
        [slab_ref[dobs:dobs + 3, 0:_RNN_H],
         slab_ref[512:513, 0:_RNN_H]], axis=0)
    wh = slab_ref[128:128 + _RNN_H, 0:_RNN_H]
    w1a = jnp.concatenate(
        [slab_ref[256:256 + _RNN_H, :], slab_ref[513:514, :]], axis=0)
    whda = jnp.concatenate(
        [slab_ref[384:512, 64:128], slab_ref[514:515, 64:128]], axis=0)

    def heads_rv(i):
        yaug = jnp.concatenate(
            [y_scr[i * 128:(i + 1) * 128, :], ones1], axis=0)
        rv = _dott(whda, yaug)
        ri_ref[i:i + 1, :] = rv[0:1, :]
        lv_ref[i:i + 1, :] = rv[1:2, :]

    h = h_scr[...]
    haug = jnp.concatenate([h, ones1], axis=0)
    for i in range(ct):
        if i >= 1:
            y_scr[(i - 1) * 128:i * 128, :] = jnp.maximum(_dott(w1a, haug), 0.0)
        if i >= 2:
            heads_rv(i - 2)
        ard = jnp.concatenate(
            [a_ref[i:i + 1, :], r_ref[i:i + 1, :], d_ref[i:i + 1, :], ones1],
            axis=0)
        px = _dott(wx, s_ref[:, i, :]) + _dott(ward, ard)
        h = jnp.tanh(px + _dott(wh, h))
        hn_ref[i] = h
        haug = jnp.concatenate([h, ones1], axis=0)
    h_scr[...] = h

    y_scr[(ct - 1) * 128:ct * 128, :] = jnp.maximum(_dott(w1a, haug), 0.0)
    for i in range(max(ct - 2, 0), ct):
        heads_rv(i)


def kernel(slab, s, a, r, d):
    s = jnp.asarray(s, jnp.float32)
    a = jnp.asarray(a, jnp.float32)
    r = jnp.asarray(r, jnp.float32)
    d = jnp.asarray(d, jnp.float32)
    T, B, Dobs = s.shape
    ct = _CT if T % _CT == 0 else 1

    st = jnp.transpose(s, (2, 0, 1))

    hn_t, ri, lv = pl.pallas_call(
        functools.partial(_fused_kernel, ct=ct),
        out_shape=(jax.ShapeDtypeStruct((T, _RNN_H, B), jnp.float32),
                   jax.ShapeDtypeStruct((T, B), jnp.float32),
                   jax.ShapeDtypeStruct((T, B), jnp.float32)),
        grid=(T // ct,),
        in_specs=[
            pl.BlockSpec((Dobs, ct, B), lambda t: (0, t, 0)),
            pl.BlockSpec((ct, B), lambda t: (t, 0)),
            pl.BlockSpec((ct, B), lambda t: (t, 0)),
            pl.BlockSpec((ct, B), lambda t: (t, 0)),
            pl.BlockSpec((520, 128), lambda t: (0, 0)),
        ],
        out_specs=(pl.BlockSpec((ct, _RNN_H, B), lambda t: (t, 0, 0)),
                   pl.BlockSpec((ct, B), lambda t: (t, 0)),
                   pl.BlockSpec((ct, B), lambda t: (t, 0))),
        scratch_shapes=[pltpu.VMEM((_RNN_H, B), jnp.float32),
                        pltpu.VMEM((ct * 128, B), jnp.float32)],
        compiler_params=pltpu.CompilerParams(
            dimension_semantics=("arbitrary",)),
    )(st, a, r, d, slab)

    hn = jnp.transpose(hn_t, (0, 2, 1))
    return ri[..., None], lv[..., None], hn
